# baseline (speedup 1.0000x reference)
import functools

import jax
import jax.numpy as jnp
from jax import lax
from jax.experimental import pallas as pl
from jax.experimental.pallas import tpu as pltpu
from jax.experimental.pallas import tpu_sc as plsc

FP = 48
NC = 2
NS = 16
LANES = 16
CB = 128
GP = 4


def _proj_body(x_ref, w_ref, d_ref, o_ref):
    z = jnp.dot(x_ref[...], w_ref[...], preferred_element_type=jnp.float32)
    deg = 1.0 + jnp.sum(d_ref[...], axis=1, keepdims=True)
    dinv = jnp.where(deg > 0, jax.lax.rsqrt(deg), 0.0)
    o_ref[...] = dinv * z


def _proj(xp, Wp, degT):
    n, f = xp.shape
    bm = n // 16
    return pl.pallas_call(
        _proj_body,
        grid=(n // bm,),
        in_specs=[
            pl.BlockSpec((bm, f), lambda i: (i, 0)),
            pl.BlockSpec((f, FP), lambda i: (0, 0)),
            pl.BlockSpec((bm, NC), lambda i: (i, 0)),
        ],
        out_specs=pl.BlockSpec((bm, FP), lambda i: (i, 0)),
        out_shape=jax.ShapeDtypeStruct((n, FP), jnp.float32),
    )(xp, Wp, degT)


def _combine_body(p0_ref, p1_ref, u_ref, d_ref, o_ref):
    deg = 1.0 + jnp.sum(d_ref[...], axis=1, keepdims=True)
    dinv2 = jnp.where(deg > 0, 1.0 / deg, 0.0)
    o_ref[...] = dinv2 * (p0_ref[...] + p1_ref[...] + u_ref[...])


def _combine(p0, p1, u, degT):
    n, fp = p0.shape
    bm = n // 16
    bs = pl.BlockSpec((bm, fp), lambda i: (i, 0))
    return pl.pallas_call(
        _combine_body,
        grid=(n // bm,),
        in_specs=[bs, bs, bs, pl.BlockSpec((bm, NC), lambda i: (i, 0))],
        out_specs=bs,
        out_shape=jax.ShapeDtypeStruct((n, fp), jnp.float32),
    )(p0, p1, u, degT)


def _final_body(nclass, q0_ref, q1_ref, u_ref, d_ref, bias_ref, o_ref):
    deg = 1.0 + jnp.sum(d_ref[...], axis=1, keepdims=True)
    dinv = jnp.where(deg > 0, jax.lax.rsqrt(deg), 0.0)
    l = dinv * (q0_ref[...] + q1_ref[...] + u_ref[...]) + bias_ref[...]
    mask = jax.lax.broadcasted_iota(jnp.int32, l.shape, 1) < nclass
    lm = jnp.where(mask, l, -jnp.inf)
    m = jnp.max(lm, axis=1, keepdims=True)
    s = jnp.sum(jnp.where(mask, jnp.exp(lm - m), 0.0), axis=1, keepdims=True)
    o_ref[...] = l - m - jnp.log(s)


def _final(q0, q1, u, degT, bp, nclass):
    n, fp = q0.shape
    bm = n // 16
    bs = pl.BlockSpec((bm, fp), lambda i: (i, 0))
    return pl.pallas_call(
        functools.partial(_final_body, nclass),
        grid=(n // bm,),
        in_specs=[bs, bs, bs,
                  pl.BlockSpec((bm, NC), lambda i: (i, 0)),
                  pl.BlockSpec((1, fp), lambda i: (0, 0))],
        out_specs=bs,
        out_shape=jax.ShapeDtypeStruct((n, fp), jnp.float32),
    )(q0, q1, u, degT, bp.reshape(1, fp))


def _lane_bcast(v, k):
    idx = jnp.full((LANES, 1), k, jnp.int32)
    dnums = lax.GatherDimensionNumbers(
        offset_dims=(), collapsed_slice_dims=(0,), start_index_map=(0,))
    return lax.gather(v, idx, dnums, (1,),
                      mode=lax.GatherScatterMode.PROMISE_IN_BOUNDS)


def _make_deg(n, cpw):
    rpt = n // NS
    mesh = plsc.VectorSubcoreMesh(core_axis_name="c", subcore_axis_name="s")

    @functools.partial(
        pl.kernel,
        out_type=jax.ShapeDtypeStruct((NC, n), jnp.float32),
        mesh=mesh,
        scratch_types=[
            pltpu.VMEM((cpw // GP, 1, GP * CB), jnp.int32),
            pltpu.VMEM((cpw // GP, 1, GP * CB), jnp.float32),
            pltpu.VMEM((CB,), jnp.float32),
            pltpu.VMEM_SHARED((n,), jnp.float32),
        ],
        compiler_params=pltpu.CompilerParams(use_tc_tiling_on_sc=False),
    )
    def deg(col_hbm, w_hbm, out_hbm, col_v, w_v, zb_v, acc_sh):
        c = lax.axis_index("c")
        s = lax.axis_index("s")
        w = c * NS + s

        pltpu.sync_copy(col_hbm.at[w], col_v)
        pltpu.sync_copy(w_hbm.at[w], w_v)

        def zs(g, _):
            zb_v[pl.ds(g * LANES, LANES)] = jnp.zeros((LANES,), jnp.float32)
            return 0
        lax.fori_loop(0, CB // LANES, zs, 0)
        r0 = s * rpt
        def zcopy(i, _):
            pltpu.sync_copy(zb_v, acc_sh.at[pl.ds(r0 + i * CB, CB)])
            return 0
        lax.fori_loop(0, rpt // CB, zcopy, 0)
        rem = rpt - (rpt // CB) * CB
        if rem:
            pltpu.sync_copy(zb_v.at[pl.ds(0, rem)],
                            acc_sh.at[pl.ds(r0 + (rpt // CB) * CB, rem)])
        plsc.subcore_barrier()

        def chunk(jg, _):
            pltpu.sync_copy(w_v.at[jg, 0], acc_sh.at[col_v.at[jg, 0]],
                            add=True)
            return 0
        lax.fori_loop(0, cpw // GP, chunk, 0)
        plsc.subcore_barrier()

        pltpu.sync_copy(acc_sh.at[pl.ds(r0, rpt)],
                        out_hbm.at[c, pl.ds(r0, rpt)])

    return deg


def _make_prop(n, cpw):
    rpt = n // NS
    assert cpw % GP == 0
    mesh = plsc.VectorSubcoreMesh(core_axis_name="c", subcore_axis_name="s")

    @functools.partial(
        pl.kernel,
        out_type=jax.ShapeDtypeStruct((NC, n, FP), jnp.float32),
        mesh=mesh,
        scratch_types=[
            pltpu.VMEM((cpw // GP, 1, GP * CB), jnp.int32),
            pltpu.VMEM((cpw // GP, 1, GP * CB), jnp.int32),
            pltpu.VMEM((cpw // GP, 1, GP * CB), jnp.float32),
            pltpu.VMEM((GP * CB, FP), jnp.float32),
            pltpu.VMEM_SHARED((n, FP), jnp.float32),
        ],
        compiler_params=pltpu.CompilerParams(use_tc_tiling_on_sc=False),
    )
    def prop(u_hbm, row_hbm, col_hbm, w_hbm, out_hbm,
             row_v, col_v, w_v, gath_a, acc_sh):
        c = lax.axis_index("c")
        s = lax.axis_index("s")
        w = c * NS + s

        pltpu.sync_copy(row_hbm.at[w], row_v)
        pltpu.sync_copy(col_hbm.at[w], col_v)
        pltpu.sync_copy(w_hbm.at[w], w_v)

        def zrow(r, _):
            for fb in range(FP // LANES):
                gath_a[r, pl.ds(fb * LANES, LANES)] = jnp.zeros(
                    (LANES,), jnp.float32)
            return 0
        lax.fori_loop(0, CB, zrow, 0)
        r0 = s * rpt
        nz = rpt // CB
        def zcopy(i, _):
            pltpu.sync_copy(gath_a.at[pl.ds(0, CB)],
                            acc_sh.at[pl.ds(r0 + i * CB, CB)])
            return 0
        lax.fori_loop(0, nz, zcopy, 0)
        rem = rpt - nz * CB
        if rem:
            pltpu.sync_copy(gath_a.at[pl.ds(0, rem)],
                            acc_sh.at[pl.ds(r0 + nz * CB, rem)])
        plsc.subcore_barrier()

        def scale2(jg, t, buf_v):
            for g in range(CB // LANES):
                wv = w_v[jg, 0, pl.ds(t * CB + g * LANES, LANES)]
                for k in range(LANES):
                    ns_ = _lane_bcast(wv, k)
                    e_ = t * CB + g * LANES + k
                    for fb in range(FP // LANES):
                        sl = pl.ds(fb * LANES, LANES)
                        buf_v[e_, sl] = buf_v[e_, sl] * ns_

        def group(jg, _):
            pltpu.sync_copy(u_hbm.at[row_v.at[jg, 0]], gath_a)
            def sub(t, _):
                scale2(jg, t, gath_a)
                return 0
            lax.fori_loop(0, GP, sub, 0)
            pltpu.sync_copy(gath_a, acc_sh.at[col_v.at[jg, 0]], add=True)
            return 0
        lax.fori_loop(0, cpw // GP, group, 0)
        plsc.subcore_barrier()

        pltpu.sync_copy(acc_sh.at[pl.ds(r0, rpt)],
                        out_hbm.at[c, pl.ds(r0, rpt)])

    return prop


def kernel(x, edge_index, edge_attr, W, b):
    n = x.shape[0]
    e = edge_attr.shape[0]
    nclass = W.shape[1]
    row = edge_index[0]
    col = edge_index[1]

    npad = -(-n // (8 * NS)) * (8 * NS)

    nw = NC * NS
    cpw = -(-e // (nw * CB))
    cpw = -(-cpw // GP) * GP
    pad = nw * cpw * CB - e
    rowp = jnp.concatenate([row, jnp.zeros((pad,), jnp.int32)])
    colp = jnp.concatenate([col, jnp.zeros((pad,), jnp.int32)])
    wp = jnp.concatenate([edge_attr, jnp.zeros((pad,), jnp.float32)])
    rowp = rowp.reshape(nw, cpw // GP, 1, GP * CB)
    colp = colp.reshape(nw, cpw // GP, 1, GP * CB)
    wp = wp.reshape(nw, cpw // GP, 1, GP * CB)

    Wp = jnp.zeros((W.shape[0], FP), W.dtype).at[:, :nclass].set(W)
    bp = jnp.zeros((FP,), b.dtype).at[:nclass].set(b)
    xp = jnp.zeros((npad, x.shape[1]), x.dtype).at[:n].set(x)

    degf = _make_deg(npad, cpw)
    prop = _make_prop(npad, cpw)

    degT = jnp.transpose(degf(colp, wp))
    u0 = _proj(xp, Wp, degT)
    p = prop(u0, rowp, colp, wp)
    u1 = _combine(p[0], p[1], u0, degT)
    q = prop(u1, rowp, colp, wp)
    out = _final(q[0], q[1], u1, degT, bp, nclass)
    return out[:n, :nclass]


# speedup vs baseline: 1.0034x; 1.0034x over previous
import functools

import jax
import jax.numpy as jnp
from jax import lax
from jax.experimental import pallas as pl
from jax.experimental.pallas import tpu as pltpu
from jax.experimental.pallas import tpu_sc as plsc

FP = 48
NC = 2
NS = 16
LANES = 16
CB = 512


def _proj_body(x_ref, w_ref, d_ref, o_ref):
    z = jnp.dot(x_ref[...], w_ref[...], preferred_element_type=jnp.float32)
    deg = 1.0 + jnp.sum(d_ref[...], axis=1, keepdims=True)
    dinv = jnp.where(deg > 0, jax.lax.rsqrt(deg), 0.0)
    o_ref[...] = dinv * z


def _proj(xp, Wp, degT):
    n, f = xp.shape
    bm = n // 16
    return pl.pallas_call(
        _proj_body,
        grid=(n // bm,),
        in_specs=[
            pl.BlockSpec((bm, f), lambda i: (i, 0)),
            pl.BlockSpec((f, FP), lambda i: (0, 0)),
            pl.BlockSpec((bm, NC), lambda i: (i, 0)),
        ],
        out_specs=pl.BlockSpec((bm, FP), lambda i: (i, 0)),
        out_shape=jax.ShapeDtypeStruct((n, FP), jnp.float32),
    )(xp, Wp, degT)


def _combine_body(p0_ref, p1_ref, u_ref, d_ref, o_ref):
    deg = 1.0 + jnp.sum(d_ref[...], axis=1, keepdims=True)
    dinv2 = jnp.where(deg > 0, 1.0 / deg, 0.0)
    o_ref[...] = dinv2 * (p0_ref[...] + p1_ref[...] + u_ref[...])


def _combine(p0, p1, u, degT):
    n, fp = p0.shape
    bm = n // 16
    bs = pl.BlockSpec((bm, fp), lambda i: (i, 0))
    return pl.pallas_call(
        _combine_body,
        grid=(n // bm,),
        in_specs=[bs, bs, bs, pl.BlockSpec((bm, NC), lambda i: (i, 0))],
        out_specs=bs,
        out_shape=jax.ShapeDtypeStruct((n, fp), jnp.float32),
    )(p0, p1, u, degT)


def _final_body(nclass, q0_ref, q1_ref, u_ref, d_ref, bias_ref, o_ref):
    deg = 1.0 + jnp.sum(d_ref[...], axis=1, keepdims=True)
    dinv = jnp.where(deg > 0, jax.lax.rsqrt(deg), 0.0)
    l = dinv * (q0_ref[...] + q1_ref[...] + u_ref[...]) + bias_ref[...]
    mask = jax.lax.broadcasted_iota(jnp.int32, l.shape, 1) < nclass
    lm = jnp.where(mask, l, -jnp.inf)
    m = jnp.max(lm, axis=1, keepdims=True)
    s = jnp.sum(jnp.where(mask, jnp.exp(lm - m), 0.0), axis=1, keepdims=True)
    o_ref[...] = l - m - jnp.log(s)


def _final(q0, q1, u, degT, bp, nclass):
    n, fp = q0.shape
    bm = n // 16
    bs = pl.BlockSpec((bm, fp), lambda i: (i, 0))
    return pl.pallas_call(
        functools.partial(_final_body, nclass),
        grid=(n // bm,),
        in_specs=[bs, bs, bs,
                  pl.BlockSpec((bm, NC), lambda i: (i, 0)),
                  pl.BlockSpec((1, fp), lambda i: (0, 0))],
        out_specs=bs,
        out_shape=jax.ShapeDtypeStruct((n, fp), jnp.float32),
    )(q0, q1, u, degT, bp.reshape(1, fp))


def _lane_bcast(v, k):
    idx = jnp.full((LANES, 1), k, jnp.int32)
    dnums = lax.GatherDimensionNumbers(
        offset_dims=(), collapsed_slice_dims=(0,), start_index_map=(0,))
    return lax.gather(v, idx, dnums, (1,),
                      mode=lax.GatherScatterMode.PROMISE_IN_BOUNDS)


def _make_deg(n, cpw):
    rpt = n // NS
    mesh = plsc.VectorSubcoreMesh(core_axis_name="c", subcore_axis_name="s")

    @functools.partial(
        pl.kernel,
        out_type=jax.ShapeDtypeStruct((NC, n), jnp.float32),
        mesh=mesh,
        scratch_types=[
            pltpu.VMEM((cpw, CB), jnp.int32),
            pltpu.VMEM((cpw, CB), jnp.float32),
            pltpu.VMEM((CB,), jnp.float32),
            pltpu.VMEM_SHARED((n,), jnp.float32),
        ],
        compiler_params=pltpu.CompilerParams(use_tc_tiling_on_sc=False),
    )
    def deg(col_hbm, w_hbm, out_hbm, col_v, w_v, zb_v, acc_sh):
        c = lax.axis_index("c")
        s = lax.axis_index("s")
        w = c * NS + s

        pltpu.sync_copy(col_hbm.at[w], col_v)
        pltpu.sync_copy(w_hbm.at[w], w_v)

        def zs(g, _):
            zb_v[pl.ds(g * LANES, LANES)] = jnp.zeros((LANES,), jnp.float32)
            return 0
        lax.fori_loop(0, CB // LANES, zs, 0)
        r0 = s * rpt
        def zcopy(i, _):
            pltpu.sync_copy(zb_v, acc_sh.at[pl.ds(r0 + i * CB, CB)])
            return 0
        lax.fori_loop(0, rpt // CB, zcopy, 0)
        rem = rpt - (rpt // CB) * CB
        if rem:
            pltpu.sync_copy(zb_v.at[pl.ds(0, rem)],
                            acc_sh.at[pl.ds(r0 + (rpt // CB) * CB, rem)])
        plsc.subcore_barrier()

        def chunk(j, _):
            pltpu.sync_copy(w_v.at[j], acc_sh.at[col_v.at[j]], add=True)
            return 0
        lax.fori_loop(0, cpw, chunk, 0)
        plsc.subcore_barrier()

        pltpu.sync_copy(acc_sh.at[pl.ds(r0, rpt)],
                        out_hbm.at[c, pl.ds(r0, rpt)])

    return deg


def _make_prop(n, cpw):
    rpt = n // NS
    mesh = plsc.VectorSubcoreMesh(core_axis_name="c", subcore_axis_name="s")

    @functools.partial(
        pl.kernel,
        out_type=jax.ShapeDtypeStruct((NC, n, FP), jnp.float32),
        mesh=mesh,
        scratch_types=[
            pltpu.VMEM((cpw, CB), jnp.int32),
            pltpu.VMEM((cpw, CB), jnp.int32),
            pltpu.VMEM((cpw, CB), jnp.float32),
            pltpu.VMEM((CB, FP), jnp.float32),
            pltpu.VMEM_SHARED((n, FP), jnp.float32),
        ],
        compiler_params=pltpu.CompilerParams(use_tc_tiling_on_sc=False),
    )
    def prop(u_hbm, row_hbm, col_hbm, w_hbm, out_hbm,
             row_v, col_v, w_v, gath_a, acc_sh):
        c = lax.axis_index("c")
        s = lax.axis_index("s")
        w = c * NS + s

        pltpu.sync_copy(row_hbm.at[w], row_v)
        pltpu.sync_copy(col_hbm.at[w], col_v)
        pltpu.sync_copy(w_hbm.at[w], w_v)

        def zrow(r, _):
            for fb in range(FP // LANES):
                gath_a[r, pl.ds(fb * LANES, LANES)] = jnp.zeros(
                    (LANES,), jnp.float32)
            return 0
        lax.fori_loop(0, CB, zrow, 0)
        r0 = s * rpt
        nz = rpt // CB
        def zcopy(i, _):
            pltpu.sync_copy(gath_a.at[pl.ds(0, CB)],
                            acc_sh.at[pl.ds(r0 + i * CB, CB)])
            return 0
        lax.fori_loop(0, nz, zcopy, 0)
        rem = rpt - nz * CB
        if rem:
            pltpu.sync_copy(gath_a.at[pl.ds(0, rem)],
                            acc_sh.at[pl.ds(r0 + nz * CB, rem)])
        plsc.subcore_barrier()

        def chunk(j, _):
            pltpu.sync_copy(u_hbm.at[row_v.at[j]], gath_a)
            def egroup(g, _):
                wv = w_v[j, pl.ds(g * LANES, LANES)]
                for k in range(LANES):
                    ns_ = _lane_bcast(wv, k)
                    e_ = g * LANES + k
                    for fb in range(FP // LANES):
                        sl = pl.ds(fb * LANES, LANES)
                        gath_a[e_, sl] = gath_a[e_, sl] * ns_
                return 0
            lax.fori_loop(0, CB // LANES, egroup, 0)
            pltpu.sync_copy(gath_a, acc_sh.at[col_v.at[j]], add=True)
            return 0
        lax.fori_loop(0, cpw, chunk, 0)
        plsc.subcore_barrier()

        pltpu.sync_copy(acc_sh.at[pl.ds(r0, rpt)],
                        out_hbm.at[c, pl.ds(r0, rpt)])

    return prop


def kernel(x, edge_index, edge_attr, W, b):
    n = x.shape[0]
    e = edge_attr.shape[0]
    nclass = W.shape[1]
    row = edge_index[0]
    col = edge_index[1]

    npad = -(-n // (8 * NS)) * (8 * NS)

    nw = NC * NS
    cpw = -(-e // (nw * CB))
    pad = nw * cpw * CB - e
    rowp = jnp.concatenate([row, jnp.zeros((pad,), jnp.int32)])
    colp = jnp.concatenate([col, jnp.zeros((pad,), jnp.int32)])
    wp = jnp.concatenate([edge_attr, jnp.zeros((pad,), jnp.float32)])
    rowp = rowp.reshape(nw, cpw, CB)
    colp = colp.reshape(nw, cpw, CB)
    wp = wp.reshape(nw, cpw, CB)

    Wp = jnp.zeros((W.shape[0], FP), W.dtype).at[:, :nclass].set(W)
    bp = jnp.zeros((FP,), b.dtype).at[:nclass].set(b)
    xp = jnp.zeros((npad, x.shape[1]), x.dtype).at[:n].set(x)

    degf = _make_deg(npad, cpw)
    prop = _make_prop(npad, cpw)

    degT = jnp.transpose(degf(colp, wp))
    u0 = _proj(xp, Wp, degT)
    p = prop(u0, rowp, colp, wp)
    u1 = _combine(p[0], p[1], u0, degT)
    q = prop(u1, rowp, colp, wp)
    out = _final(q[0], q[1], u1, degT, bp, nclass)
    return out[:n, :nclass]


# speedup vs baseline: 1.2818x; 1.2774x over previous
import functools

import jax
import jax.numpy as jnp
from jax import lax
from jax.experimental import pallas as pl
from jax.experimental.pallas import tpu as pltpu
from jax.experimental.pallas import tpu_sc as plsc

FP = 48
NC = 2
NS = 16
LANES = 16
CB = 128


def _proj_body(x_ref, w_ref, d_ref, o_ref):
    z = jnp.dot(x_ref[...], w_ref[...], preferred_element_type=jnp.float32)
    deg = 1.0 + jnp.sum(d_ref[...], axis=1, keepdims=True)
    dinv = jnp.where(deg > 0, jax.lax.rsqrt(deg), 0.0)
    o_ref[...] = dinv * z


def _proj(xp, Wp, degT):
    n, f = xp.shape
    bm = n // 16
    return pl.pallas_call(
        _proj_body,
        grid=(n // bm,),
        in_specs=[
            pl.BlockSpec((bm, f), lambda i: (i, 0)),
            pl.BlockSpec((f, FP), lambda i: (0, 0)),
            pl.BlockSpec((bm, NC), lambda i: (i, 0)),
        ],
        out_specs=pl.BlockSpec((bm, FP), lambda i: (i, 0)),
        out_shape=jax.ShapeDtypeStruct((n, FP), jnp.float32),
    )(xp, Wp, degT)


def _combine_body(p0_ref, p1_ref, u_ref, d_ref, o_ref):
    deg = 1.0 + jnp.sum(d_ref[...], axis=1, keepdims=True)
    dinv2 = jnp.where(deg > 0, 1.0 / deg, 0.0)
    o_ref[...] = dinv2 * (p0_ref[...] + p1_ref[...] + u_ref[...])


def _combine(p0, p1, u, degT):
    n, fp = p0.shape
    bm = n // 16
    bs = pl.BlockSpec((bm, fp), lambda i: (i, 0))
    return pl.pallas_call(
        _combine_body,
        grid=(n // bm,),
        in_specs=[bs, bs, bs, pl.BlockSpec((bm, NC), lambda i: (i, 0))],
        out_specs=bs,
        out_shape=jax.ShapeDtypeStruct((n, fp), jnp.float32),
    )(p0, p1, u, degT)


def _final_body(nclass, q0_ref, q1_ref, u_ref, d_ref, bias_ref, o_ref):
    deg = 1.0 + jnp.sum(d_ref[...], axis=1, keepdims=True)
    dinv = jnp.where(deg > 0, jax.lax.rsqrt(deg), 0.0)
    l = dinv * (q0_ref[...] + q1_ref[...] + u_ref[...]) + bias_ref[...]
    mask = jax.lax.broadcasted_iota(jnp.int32, l.shape, 1) < nclass
    lm = jnp.where(mask, l, -jnp.inf)
    m = jnp.max(lm, axis=1, keepdims=True)
    s = jnp.sum(jnp.where(mask, jnp.exp(lm - m), 0.0), axis=1, keepdims=True)
    o_ref[...] = l - m - jnp.log(s)


def _final(q0, q1, u, degT, bp, nclass):
    n, fp = q0.shape
    bm = n // 16
    bs = pl.BlockSpec((bm, fp), lambda i: (i, 0))
    return pl.pallas_call(
        functools.partial(_final_body, nclass),
        grid=(n // bm,),
        in_specs=[bs, bs, bs,
                  pl.BlockSpec((bm, NC), lambda i: (i, 0)),
                  pl.BlockSpec((1, fp), lambda i: (0, 0))],
        out_specs=bs,
        out_shape=jax.ShapeDtypeStruct((n, fp), jnp.float32),
    )(q0, q1, u, degT, bp.reshape(1, fp))


def _lane_bcast(v, k):
    idx = jnp.full((LANES, 1), k, jnp.int32)
    dnums = lax.GatherDimensionNumbers(
        offset_dims=(), collapsed_slice_dims=(0,), start_index_map=(0,))
    return lax.gather(v, idx, dnums, (1,),
                      mode=lax.GatherScatterMode.PROMISE_IN_BOUNDS)


def _make_deg(n, cpw):
    rpt = n // NS
    mesh = plsc.VectorSubcoreMesh(core_axis_name="c", subcore_axis_name="s")

    @functools.partial(
        pl.kernel,
        out_type=jax.ShapeDtypeStruct((NC, n), jnp.float32),
        mesh=mesh,
        scratch_types=[
            pltpu.VMEM((cpw, CB), jnp.int32),
            pltpu.VMEM((cpw, CB), jnp.float32),
            pltpu.VMEM((CB,), jnp.float32),
            pltpu.VMEM_SHARED((n,), jnp.float32),
        ],
        compiler_params=pltpu.CompilerParams(use_tc_tiling_on_sc=False),
    )
    def deg(col_hbm, w_hbm, out_hbm, col_v, w_v, zb_v, acc_sh):
        c = lax.axis_index("c")
        s = lax.axis_index("s")
        w = c * NS + s

        pltpu.sync_copy(col_hbm.at[w], col_v)
        pltpu.sync_copy(w_hbm.at[w], w_v)

        def zs(g, _):
            zb_v[pl.ds(g * LANES, LANES)] = jnp.zeros((LANES,), jnp.float32)
            return 0
        lax.fori_loop(0, CB // LANES, zs, 0)
        r0 = s * rpt
        def zcopy(i, _):
            pltpu.sync_copy(zb_v, acc_sh.at[pl.ds(r0 + i * CB, CB)])
            return 0
        lax.fori_loop(0, rpt // CB, zcopy, 0)
        rem = rpt - (rpt // CB) * CB
        if rem:
            pltpu.sync_copy(zb_v.at[pl.ds(0, rem)],
                            acc_sh.at[pl.ds(r0 + (rpt // CB) * CB, rem)])
        plsc.subcore_barrier()

        def chunk(j, _):
            pltpu.sync_copy(w_v.at[j], acc_sh.at[col_v.at[j]], add=True)
            return 0
        lax.fori_loop(0, cpw, chunk, 0)
        plsc.subcore_barrier()

        pltpu.sync_copy(acc_sh.at[pl.ds(r0, rpt)],
                        out_hbm.at[c, pl.ds(r0, rpt)])

    return deg


def _make_prop(n, cpw):
    rpt = n // NS
    mesh = plsc.VectorSubcoreMesh(core_axis_name="c", subcore_axis_name="s")

    @functools.partial(
        pl.kernel,
        out_type=jax.ShapeDtypeStruct((NC, n, FP), jnp.float32),
        mesh=mesh,
        scratch_types=[
            pltpu.VMEM((cpw, CB), jnp.int32),
            pltpu.VMEM((cpw, CB), jnp.int32),
            pltpu.VMEM((cpw, CB), jnp.float32),
            pltpu.VMEM((CB, FP), jnp.float32),
            pltpu.VMEM_SHARED((n, FP), jnp.float32),
        ],
        compiler_params=pltpu.CompilerParams(use_tc_tiling_on_sc=False),
    )
    def prop(u_hbm, row_hbm, col_hbm, w_hbm, out_hbm,
             row_v, col_v, w_v, gath_a, acc_sh):
        c = lax.axis_index("c")
        s = lax.axis_index("s")
        w = c * NS + s

        pltpu.sync_copy(row_hbm.at[w], row_v)
        pltpu.sync_copy(col_hbm.at[w], col_v)
        pltpu.sync_copy(w_hbm.at[w], w_v)

        def zrow(r, _):
            for fb in range(FP // LANES):
                gath_a[r, pl.ds(fb * LANES, LANES)] = jnp.zeros(
                    (LANES,), jnp.float32)
            return 0
        lax.fori_loop(0, CB, zrow, 0)
        r0 = s * rpt
        nz = rpt // CB
        def zcopy(i, _):
            pltpu.sync_copy(gath_a.at[pl.ds(0, CB)],
                            acc_sh.at[pl.ds(r0 + i * CB, CB)])
            return 0
        lax.fori_loop(0, nz, zcopy, 0)
        rem = rpt - nz * CB
        if rem:
            pltpu.sync_copy(gath_a.at[pl.ds(0, rem)],
                            acc_sh.at[pl.ds(r0 + nz * CB, rem)])
        plsc.subcore_barrier()

        def chunk(j, _):
            pltpu.sync_copy(u_hbm.at[row_v.at[j]], gath_a)
            def egroup(g, _):
                wv = w_v[j, pl.ds(g * LANES, LANES)]
                for k in range(LANES):
                    ns_ = _lane_bcast(wv, k)
                    e_ = g * LANES + k
                    for fb in range(FP // LANES):
                        sl = pl.ds(fb * LANES, LANES)
                        gath_a[e_, sl] = gath_a[e_, sl] * ns_
                return 0
            lax.fori_loop(0, CB // LANES, egroup, 0)
            pltpu.sync_copy(gath_a, acc_sh.at[col_v.at[j]], add=True)
            return 0
        lax.fori_loop(0, cpw, chunk, 0)
        plsc.subcore_barrier()

        pltpu.sync_copy(acc_sh.at[pl.ds(r0, rpt)],
                        out_hbm.at[c, pl.ds(r0, rpt)])

    return prop


def kernel(x, edge_index, edge_attr, W, b):
    n = x.shape[0]
    e = edge_attr.shape[0]
    nclass = W.shape[1]
    row = edge_index[0]
    col = edge_index[1]

    npad = -(-n // (8 * NS)) * (8 * NS)

    nw = NC * NS
    cpw = -(-e // (nw * CB))
    pad = nw * cpw * CB - e
    rowp = jnp.concatenate([row, jnp.zeros((pad,), jnp.int32)])
    colp = jnp.concatenate([col, jnp.zeros((pad,), jnp.int32)])
    wp = jnp.concatenate([edge_attr, jnp.zeros((pad,), jnp.float32)])
    rowp = rowp.reshape(nw, cpw, CB)
    colp = colp.reshape(nw, cpw, CB)
    wp = wp.reshape(nw, cpw, CB)

    Wp = jnp.zeros((W.shape[0], FP), W.dtype).at[:, :nclass].set(W)
    bp = jnp.zeros((FP,), b.dtype).at[:nclass].set(b)
    xp = jnp.zeros((npad, x.shape[1]), x.dtype).at[:n].set(x)

    degf = _make_deg(npad, cpw)
    prop = _make_prop(npad, cpw)

    degT = jnp.transpose(degf(colp, wp))
    u0 = _proj(xp, Wp, degT)
    p = prop(u0, rowp, colp, wp)
    u1 = _combine(p[0], p[1], u0, degT)
    q = prop(u1, rowp, colp, wp)
    out = _final(q[0], q[1], u1, degT, bp, nclass)
    return out[:n, :nclass]
